# Initial kernel scaffold; baseline (speedup 1.0000x reference)
import jax
import jax.numpy as jnp
from jax.experimental import pallas as pl


def kernel(features, vertices, edges, faces, Wfc0, bfc0, T1, W1, b1, T2, W2, b2, T3, W3, b3, Wfc1, bfc1, Wfc2, bfc2):
    raise NotImplementedError("write your pallas kernel here")


# speedup vs baseline: 1.0971x; 1.0971x over previous
import functools

import jax
import jax.numpy as jnp
from jax.experimental import pallas as pl


def _elu(x):
    return jnp.where(x > 0, x, jnp.exp(jnp.minimum(x, 0.0)) - 1.0)


def _dense(x, W, b, act: bool, blk: int = 2000):
    N, Cin = x.shape
    Cout = W.shape[0]

    def body(x_ref, w_ref, b_ref, o_ref):
        y = jnp.dot(x_ref[...], w_ref[...].T, preferred_element_type=jnp.float32)
        y = y + b_ref[...]
        o_ref[...] = _elu(y) if act else y

    return pl.pallas_call(
        body,
        grid=(N // blk,),
        in_specs=[
            pl.BlockSpec((blk, Cin), lambda i: (i, 0)),
            pl.BlockSpec((Cout, Cin), lambda i: (0, 0)),
            pl.BlockSpec((1, Cout), lambda i: (0, 0)),
        ],
        out_specs=pl.BlockSpec((blk, Cout), lambda i: (i, 0)),
        out_shape=jax.ShapeDtypeStruct((N, Cout), jnp.float32),
    )(x, W, b.reshape(1, Cout))


def _metric_conv(x, vertices, src, dst, T, W, b):
    d = vertices[src] - vertices[dst]
    Td = d @ T.T
    dist = jnp.sum(Td * Td, axis=-1)
    w_e = jnp.exp(-dist)
    msg = w_e[:, None] * x[src]
    agg = jax.ops.segment_sum(msg, dst, num_segments=x.shape[0])
    denom = jax.ops.segment_sum(w_e, dst, num_segments=x.shape[0]) + 1.0
    h = (agg + x) / denom[:, None]
    return _dense(h, W, b, act=True)


def kernel(features, vertices, edges, faces, Wfc0, bfc0, T1, W1, b1, T2, W2, b2, T3, W3, b3, Wfc1, bfc1, Wfc2, bfc2):
    src = edges[0]
    dst = edges[1]
    x = _dense(features, Wfc0, bfc0, act=True)
    x = _metric_conv(x, vertices, src, dst, T1, W1, b1)
    x = _metric_conv(x, vertices, src, dst, T2, W2, b2)
    x = _metric_conv(x, vertices, src, dst, T3, W3, b3)
    x = _dense(x, Wfc1, bfc1, act=True)
    out = _dense(x, Wfc2, bfc2, act=False)
    return out


# speedup vs baseline: 1.5473x; 1.4103x over previous
import functools

import jax
import jax.numpy as jnp
from jax import lax
from jax.experimental import pallas as pl
from jax.experimental.pallas import tpu as pltpu
from jax.experimental.pallas import tpu_sc as plsc

N = 100000
E = 1600000
NP = 102400
NC = 2
NS = 16
NW = NC * NS
EPT = E // NW
K = 2000
NCH = EPT // K


def _elu(x):
    return jnp.where(x > 0, x, jnp.exp(jnp.minimum(x, 0.0)) - 1.0)


def _dense(x, W, b, act: bool, blk: int = 2000):
    n, Cin = x.shape
    Cout = W.shape[0]

    def body(x_ref, w_ref, b_ref, o_ref):
        y = jnp.dot(x_ref[...], w_ref[...].T, preferred_element_type=jnp.float32)
        y = y + b_ref[...]
        o_ref[...] = _elu(y) if act else y

    return pl.pallas_call(
        body,
        grid=(n // blk,),
        in_specs=[
            pl.BlockSpec((blk, Cin), lambda i: (i, 0)),
            pl.BlockSpec((Cout, Cin), lambda i: (0, 0)),
            pl.BlockSpec((1, Cout), lambda i: (0, 0)),
        ],
        out_specs=pl.BlockSpec((blk, Cout), lambda i: (i, 0)),
        out_shape=jax.ShapeDtypeStruct((n, Cout), jnp.float32),
    )(x, W, b.reshape(1, Cout))


def _splat(ref, idx):
    return plsc.load_gather(ref, [jnp.full((16,), idx, jnp.int32)])


def _sc_weights(src, dst, vx, vy, vz, coefs, zeros_np):
    mesh = plsc.VectorSubcoreMesh(
        core_axis_name="c", subcore_axis_name="s", num_cores=NC, num_subcores=NS
    )

    @functools.partial(
        pl.kernel,
        out_type=(
            jax.ShapeDtypeStruct((E,), jnp.float32),
            jax.ShapeDtypeStruct((E,), jnp.float32),
            jax.ShapeDtypeStruct((E,), jnp.float32),
            jax.ShapeDtypeStruct((NC * 3 * NP,), jnp.float32),
        ),
        mesh=mesh,
        scratch_types=[
            pltpu.VMEM((K,), jnp.int32),
            pltpu.VMEM((K,), jnp.int32),
            pltpu.VMEM((K,), jnp.float32),
            pltpu.VMEM((K,), jnp.float32),
            pltpu.VMEM((K,), jnp.float32),
            pltpu.VMEM((K,), jnp.float32),
            pltpu.VMEM((K,), jnp.float32),
            pltpu.VMEM((K,), jnp.float32),
            pltpu.VMEM((K,), jnp.float32),
            pltpu.VMEM((K,), jnp.float32),
            pltpu.VMEM((K,), jnp.float32),
            pltpu.VMEM((18, 16), jnp.float32),
            pltpu.VMEM_SHARED((NP,), jnp.float32),
            pltpu.VMEM_SHARED((NP,), jnp.float32),
            pltpu.VMEM_SHARED((NP,), jnp.float32),
            pltpu.SemaphoreType.DMA,
        ],
    )
    def k(src_h, dst_h, vx_h, vy_h, vz_h, coef_h, zeros_h,
          w1_h, w2_h, w3_h, den_h,
          src_v, dst_v, sx, sy, sz, tx, ty, tz, w1v, w2v, w3v, coef_v,
          d1, d2, d3, sem):
        c = lax.axis_index("c")
        s = lax.axis_index("s")
        wid = c * NS + s
        t_np = s * (NP // NS)

        pltpu.sync_copy(coef_h, coef_v)
        for d in (d1, d2, d3):
            pltpu.sync_copy(zeros_h.at[pl.ds(t_np, NP // NS)],
                            d.at[pl.ds(t_np, NP // NS)])
        plsc.subcore_barrier()

        cs = [[coef_v[6 * m + j, :] for j in range(6)] for m in range(3)]

        @pl.loop(0, NCH)
        def _chunk(j):
            off = wid * EPT + j * K
            pltpu.sync_copy(src_h.at[pl.ds(off, K)], src_v)
            pltpu.sync_copy(dst_h.at[pl.ds(off, K)], dst_v)
            g1 = pltpu.async_copy(vx_h.at[src_v], sx, sem)
            g2 = pltpu.async_copy(vy_h.at[src_v], sy, sem)
            g3 = pltpu.async_copy(vz_h.at[src_v], sz, sem)
            g4 = pltpu.async_copy(vx_h.at[dst_v], tx, sem)
            g5 = pltpu.async_copy(vy_h.at[dst_v], ty, sem)
            g6 = pltpu.async_copy(vz_h.at[dst_v], tz, sem)
            for g in (g1, g2, g3, g4, g5, g6):
                g.wait()

            @pl.loop(0, K // 16)
            def _vec(i):
                sl = pl.ds(i * 16, 16)
                dx = sx[sl] - tx[sl]
                dy = sy[sl] - ty[sl]
                dz = sz[sl] - tz[sl]
                p = (dx * dx, dy * dy, dz * dz, dx * dy, dx * dz, dy * dz)
                for m, wv in enumerate((w1v, w2v, w3v)):
                    dist = cs[m][0] * p[0]
                    for q in range(1, 6):
                        dist = dist + cs[m][q] * p[q]
                    wv[sl] = jnp.exp(-dist)

            pltpu.sync_copy(w1v, w1_h.at[pl.ds(off, K)])
            pltpu.sync_copy(w2v, w2_h.at[pl.ds(off, K)])
            pltpu.sync_copy(w3v, w3_h.at[pl.ds(off, K)])
            pltpu.sync_copy(w1v, d1.at[dst_v], add=True)
            pltpu.sync_copy(w2v, d2.at[dst_v], add=True)
            pltpu.sync_copy(w3v, d3.at[dst_v], add=True)

        plsc.subcore_barrier()
        for m, d in enumerate((d1, d2, d3)):
            pltpu.sync_copy(d.at[pl.ds(t_np, NP // NS)],
                            den_h.at[pl.ds((c * 3 + m) * NP + t_np, NP // NS)])

    return k(src, dst, vx, vy, vz, coefs, zeros_np)


def _metric_conv(x, src, dst, w_e, denom, W, b):
    msg = w_e[:, None] * x[src]
    agg = jax.ops.segment_sum(msg, dst, num_segments=x.shape[0])
    h = (agg + x) / denom[:, None]
    return _dense(h, W, b, act=True)


def kernel(features, vertices, edges, faces, Wfc0, bfc0, T1, W1, b1, T2, W2, b2, T3, W3, b3, Wfc1, bfc1, Wfc2, bfc2):
    src = edges[0]
    dst = edges[1]
    vx = vertices[:, 0] + 0.0
    vy = vertices[:, 1] + 0.0
    vz = vertices[:, 2] + 0.0

    def mcoef(T):
        M = T.T @ T
        return jnp.stack([M[0, 0], M[1, 1], M[2, 2],
                          2 * M[0, 1], 2 * M[0, 2], 2 * M[1, 2]])

    coefs = jnp.concatenate([mcoef(T1), mcoef(T2), mcoef(T3)])
    coefs = jnp.tile(coefs[:, None], (1, 16))
    zeros_np = jnp.zeros((NP,), jnp.float32)

    w1, w2, w3, den_part = _sc_weights(src, dst, vx, vy, vz, coefs, zeros_np)
    den_part = den_part.reshape(NC, 3, NP)
    den = den_part[0, :, :N] + den_part[1, :, :N] + 1.0

    x = _dense(features, Wfc0, bfc0, act=True)
    x = _metric_conv(x, src, dst, w1, den[0], W1, b1)
    x = _metric_conv(x, src, dst, w2, den[1], W2, b2)
    x = _metric_conv(x, src, dst, w3, den[2], W3, b3)
    x = _dense(x, Wfc1, bfc1, act=True)
    out = _dense(x, Wfc2, bfc2, act=False)
    return out


# speedup vs baseline: 7.4893x; 4.8403x over previous
import functools

import jax
import jax.numpy as jnp
from jax import lax
from jax.experimental import pallas as pl
from jax.experimental.pallas import tpu as pltpu
from jax.experimental.pallas import tpu_sc as plsc

N = 100000
E = 1600000
EP = 1638400
NP = 102400
NC = 2
NS = 16
NW = NC * NS
KA = 2560
KC = 1280
NPT = NP // NS


def _elu(x):
    return jnp.where(x > 0, x, jnp.exp(jnp.minimum(x, 0.0)) - 1.0)


def _dense(x, W, b, act: bool, blk: int = 2048):
    n, Cin = x.shape
    Cout = W.shape[0]

    def body(x_ref, w_ref, b_ref, o_ref):
        y = jnp.dot(x_ref[...], w_ref[...].T, preferred_element_type=jnp.float32)
        y = y + b_ref[...]
        o_ref[...] = _elu(y) if act else y

    return pl.pallas_call(
        body,
        grid=(n // blk,),
        in_specs=[
            pl.BlockSpec((blk, Cin), lambda i: (i, 0)),
            pl.BlockSpec((Cout, Cin), lambda i: (0, 0)),
            pl.BlockSpec((1, Cout), lambda i: (0, 0)),
        ],
        out_specs=pl.BlockSpec((blk, Cout), lambda i: (i, 0)),
        out_shape=jax.ShapeDtypeStruct((n, Cout), jnp.float32),
    )(x, W, b.reshape(1, Cout))


def _mesh():
    return plsc.VectorSubcoreMesh(
        core_axis_name="c", subcore_axis_name="s", num_cores=NC, num_subcores=NS
    )


def _sc_weights(src2, dst2, vx, vy, vz, coefs, zeros_np):
    ept = EP // NW
    nch = ept // KA
    RA = KA // 128

    @functools.partial(
        pl.kernel,
        out_type=(
            jax.ShapeDtypeStruct((EP,), jnp.float32),
            jax.ShapeDtypeStruct((EP,), jnp.float32),
            jax.ShapeDtypeStruct((EP,), jnp.float32),
            jax.ShapeDtypeStruct((NC * 3 * NP,), jnp.float32),
        ),
        mesh=_mesh(),
        scratch_types=[
            pltpu.VMEM((RA, 1, 128), jnp.int32),
            pltpu.VMEM((RA, 1, 128), jnp.int32),
            pltpu.VMEM((KA,), jnp.float32),
            pltpu.VMEM((KA,), jnp.float32),
            pltpu.VMEM((KA,), jnp.float32),
            pltpu.VMEM((KA,), jnp.float32),
            pltpu.VMEM((KA,), jnp.float32),
            pltpu.VMEM((KA,), jnp.float32),
            pltpu.VMEM((KA,), jnp.float32),
            pltpu.VMEM((KA,), jnp.float32),
            pltpu.VMEM((KA,), jnp.float32),
            pltpu.VMEM((18, 16), jnp.float32),
            pltpu.VMEM_SHARED((NP,), jnp.float32),
            pltpu.VMEM_SHARED((NP,), jnp.float32),
            pltpu.VMEM_SHARED((NP,), jnp.float32),
            pltpu.SemaphoreType.DMA,
            pltpu.SemaphoreType.DMA,
        ],
    )
    def k(src_h, dst_h, vx_h, vy_h, vz_h, coef_h, zeros_h,
          w1_h, w2_h, w3_h, den_h,
          si2, di2, sx, sy, sz, tx, ty, tz, w1v, w2v, w3v, coef_v,
          d1, d2, d3, sem, sem2):
        c = lax.axis_index("c")
        s = lax.axis_index("s")
        wid = c * NS + s
        t_np = s * NPT

        pltpu.sync_copy(coef_h, coef_v)
        for d in (d1, d2, d3):
            pltpu.sync_copy(zeros_h.at[pl.ds(t_np, NPT)], d.at[pl.ds(t_np, NPT)])
        plsc.subcore_barrier()

        cs = [[coef_v[6 * m + j, :] for j in range(6)] for m in range(3)]

        @pl.loop(0, nch)
        def _chunk(j):
            off = wid * ept + j * KA
            row0 = (wid * ept + j * KA) // 128
            pltpu.sync_copy(src_h.at[pl.ds(row0, RA)], si2)
            pltpu.sync_copy(dst_h.at[pl.ds(row0, RA)], di2)
            gs = []
            for r in range(RA):
                sl = pl.ds(r * 128, 128)
                gs.append(pltpu.async_copy(vx_h.at[si2.at[r, 0]], sx.at[sl], sem))
                gs.append(pltpu.async_copy(vy_h.at[si2.at[r, 0]], sy.at[sl], sem))
                gs.append(pltpu.async_copy(vz_h.at[si2.at[r, 0]], sz.at[sl], sem))
                gs.append(pltpu.async_copy(vx_h.at[di2.at[r, 0]], tx.at[sl], sem))
                gs.append(pltpu.async_copy(vy_h.at[di2.at[r, 0]], ty.at[sl], sem))
                gs.append(pltpu.async_copy(vz_h.at[di2.at[r, 0]], tz.at[sl], sem))
            for g in gs:
                g.wait()

            @pl.loop(0, KA // 16)
            def _vec(i):
                sl = pl.ds(i * 16, 16)
                dx = sx[sl] - tx[sl]
                dy = sy[sl] - ty[sl]
                dz = sz[sl] - tz[sl]
                p = (dx * dx, dy * dy, dz * dz, dx * dy, dx * dz, dy * dz)
                for m, wv in enumerate((w1v, w2v, w3v)):
                    dist = cs[m][0] * p[0]
                    for q in range(1, 6):
                        dist = dist + cs[m][q] * p[q]
                    wv[sl] = jnp.exp(-dist)

            pltpu.sync_copy(w1v, w1_h.at[pl.ds(off, KA)])
            pltpu.sync_copy(w2v, w2_h.at[pl.ds(off, KA)])
            pltpu.sync_copy(w3v, w3_h.at[pl.ds(off, KA)])
            ss = []
            for r in range(RA):
                sl = pl.ds(r * 128, 128)
                ss.append(pltpu.async_copy(w1v.at[sl], d1.at[di2.at[r, 0]], sem2, add=True))
                ss.append(pltpu.async_copy(w2v.at[sl], d2.at[di2.at[r, 0]], sem2, add=True))
                ss.append(pltpu.async_copy(w3v.at[sl], d3.at[di2.at[r, 0]], sem2, add=True))
            for g in ss:
                g.wait()

        plsc.subcore_barrier()
        for m, d in enumerate((d1, d2, d3)):
            pltpu.sync_copy(d.at[pl.ds(t_np, NPT)],
                            den_h.at[pl.ds((c * 3 + m) * NP + t_np, NPT)])

    return k(src2, dst2, vx, vy, vz, coefs, zeros_np)


def _sc_conv(xg, src2, dst2, w_e, zeros16, G):
    S = 4 if G == 4 else 2
    R = 2 if G == 4 else 1
    ec = EP // NW if G == 1 else EP // NS
    nch = ec // KC
    RC = KC // 128

    @functools.partial(
        pl.kernel,
        out_type=jax.ShapeDtypeStruct((S * NP, 16), jnp.float32),
        mesh=_mesh(),
        compiler_params=pltpu.CompilerParams(use_tc_tiling_on_sc=False),
        scratch_types=[
            pltpu.VMEM((RC, 1, 128), jnp.int32),
            pltpu.VMEM((RC, 1, 128), jnp.int32),
            pltpu.VMEM((KC,), jnp.float32),
            pltpu.VMEM((KC, 16), jnp.float32),
            pltpu.VMEM_SHARED((NP, 16), jnp.float32),
            pltpu.SemaphoreType.DMA,
        ],
    )
    def k(xg_h, src_h, dst_h, w_h, z_h, out_h,
          si2, di2, w_v, rows_v, agg_sh, sem):
        c = lax.axis_index("c")
        s = lax.axis_index("s")
        rowz = s * NPT

        for r in range(R):
            g = r * NC + c
            slot = c if G == 1 else g
            pltpu.sync_copy(z_h, agg_sh.at[pl.ds(rowz, NPT)])
            plsc.subcore_barrier()

            @pl.loop(0, nch)
            def _chunk(j):
                eb = (c * NS + s) * ec if G == 1 else s * ec
                off = eb + j * KC
                row0 = off // 128
                pltpu.sync_copy(src_h.at[pl.ds(row0, RC)], si2)
                pltpu.sync_copy(dst_h.at[pl.ds(row0, RC)], di2)
                pltpu.sync_copy(w_h.at[pl.ds(off, KC)], w_v)
                if G > 1:
                    goff = jnp.full((16,), g * NP, jnp.int32)

                    @pl.loop(0, RC)
                    def _adj(rr):
                        for l in range(8):
                            sl = pl.ds(l * 16, 16)
                            si2[rr, 0, sl] = si2[rr, 0, sl] + goff

                gs = []
                for rr in range(RC):
                    gs.append(pltpu.async_copy(
                        xg_h.at[si2.at[rr, 0]],
                        rows_v.at[pl.ds(rr * 128, 128)], sem))
                for gg in gs:
                    gg.wait()

                @pl.loop(0, KC // 16)
                def _scale(jj):
                    w16 = w_v[pl.ds(jj * 16, 16)]
                    base = jj * 16
                    for el in range(16):
                        spl = w16.at[jnp.full((16,), el, jnp.int32)].get(
                            mode="promise_in_bounds")
                        rows_v[base + el, :] = rows_v[base + el, :] * spl

                ss = []
                for rr in range(RC):
                    ss.append(pltpu.async_copy(
                        rows_v.at[pl.ds(rr * 128, 128)],
                        agg_sh.at[di2.at[rr, 0]], sem, add=True))
                for gg in ss:
                    gg.wait()

            plsc.subcore_barrier()
            pltpu.sync_copy(agg_sh.at[pl.ds(rowz, NPT)],
                            out_h.at[pl.ds(slot * NP + rowz, NPT)])
            if r + 1 < R:
                plsc.subcore_barrier()

    return k(xg, src2, dst2, w_e, zeros16)


def _conv_epi(aggs, x_prev, den2, W, b, partial: bool, Gout, blk: int = 2048):
    S = aggs.shape[0]
    G = x_prev.shape[0]
    Cout = W.shape[0]

    def body(a_ref, x_ref, d_ref, w_ref, b_ref, o_ref):
        d = d_ref[...]
        inv = 1.0 / (d[:, 0] + d[:, 1] + 1.0)
        a = a_ref[...]
        xp = x_ref[...]
        glist = [a[0] + a[1]] if partial else [a[i] for i in range(S)]
        acc = None
        for g in range(G):
            h = (glist[g] + xp[g]) * inv[:, None]
            pg = jnp.dot(h, w_ref[...][:, g * 16:(g + 1) * 16].T,
                         preferred_element_type=jnp.float32)
            acc = pg if acc is None else acc + pg
        y = _elu(acc + b_ref[...])
        if Gout is None:
            o_ref[...] = y
        else:
            for go in range(Gout):
                o_ref[go] = y[:, go * 16:(go + 1) * 16]

    out_shape = (NP, Cout) if Gout is None else (Gout, NP, 16)
    out_block = (blk, Cout) if Gout is None else (Gout, blk, 16)
    out_map = (lambda i: (i, 0)) if Gout is None else (lambda i: (0, i, 0))
    return pl.pallas_call(
        body,
        grid=(NP // blk,),
        in_specs=[
            pl.BlockSpec((S, blk, 16), lambda i: (0, i, 0)),
            pl.BlockSpec((G, blk, 16), lambda i: (0, i, 0)),
            pl.BlockSpec((blk, 2), lambda i: (i, 0)),
            pl.BlockSpec((Cout, G * 16), lambda i: (0, 0)),
            pl.BlockSpec((1, Cout), lambda i: (0, 0)),
        ],
        out_specs=pl.BlockSpec(out_block, out_map),
        out_shape=jax.ShapeDtypeStruct(out_shape, jnp.float32),
    )(aggs, x_prev, den2, W, b.reshape(1, Cout))


def kernel(features, vertices, edges, faces, Wfc0, bfc0, T1, W1, b1, T2, W2, b2, T3, W3, b3, Wfc1, bfc1, Wfc2, bfc2):
    src = edges[0]
    dst = edges[1]
    pad = EP - E
    src2 = jnp.concatenate([src, jnp.zeros((pad,), jnp.int32)]).reshape(EP // 128, 1, 128)
    dst2 = jnp.concatenate([dst, jnp.full((pad,), N, jnp.int32)]).reshape(EP // 128, 1, 128)
    vpad = jnp.zeros((NP, 3), jnp.float32).at[:N].set(vertices)
    vx = vpad[:, 0] + 0.0
    vy = vpad[:, 1] + 0.0
    vz = vpad[:, 2] + 0.0

    def mcoef(T):
        M = T.T @ T
        return jnp.stack([M[0, 0], M[1, 1], M[2, 2],
                          2 * M[0, 1], 2 * M[0, 2], 2 * M[1, 2]])

    coefs = jnp.concatenate([mcoef(T1), mcoef(T2), mcoef(T3)])
    coefs = jnp.tile(coefs[:, None], (1, 16))
    zeros_np = jnp.zeros((NP,), jnp.float32)
    zeros16 = jnp.zeros((NPT, 16), jnp.float32)

    w1, w2, w3, den_part = _sc_weights(src2, dst2, vx, vy, vz, coefs, zeros_np)
    denT = jnp.transpose(den_part.reshape(NC, 3, NP), (1, 2, 0))

    feats_p = jnp.zeros((NP, 3), jnp.float32).at[:N].set(features)
    x0 = _dense(feats_p, Wfc0, bfc0, act=True)

    agg1 = _sc_conv(x0, src2, dst2, w1, zeros16, G=1)
    x1 = _conv_epi(agg1.reshape(2, NP, 16), x0.reshape(1, NP, 16),
                   denT[0], W1, b1, partial=True, Gout=2)

    agg2 = _sc_conv(x1.reshape(2 * NP, 16), src2, dst2, w2, zeros16, G=2)
    x2 = _conv_epi(agg2.reshape(2, NP, 16), x1,
                   denT[1], W2, b2, partial=False, Gout=4)

    agg3 = _sc_conv(x2.reshape(4 * NP, 16), src2, dst2, w3, zeros16, G=4)
    x3 = _conv_epi(agg3.reshape(4, NP, 16), x2,
                   denT[2], W3, b3, partial=False, Gout=None)

    x4 = _dense(x3, Wfc1, bfc1, act=True)
    out = _dense(x4, Wfc2, bfc2, act=False)
    return out[:N]
